# baseline (speedup 1.0000x reference)
import functools

import jax
import jax.numpy as jnp
from jax import lax
from jax.experimental import pallas as pl
from jax.experimental.pallas import tpu as pltpu
from jax.experimental.pallas import tpu_sc as plsc

N = 10000
M = 10000
E = 320000
G = 4
T = 3
FEAT = 512
HID = 256
B = 2048

R = 10240
RPT = R // 16
R2 = R
EPT = E // 16
CH = 80
NPAIR = 62
CHC = 2000
NCHC = EPT // CHC
L = 16

_MESH = plsc.VectorSubcoreMesh(core_axis_name="c", subcore_axis_name="s",
                               num_cores=2, num_subcores=16)


def _fill_loop(ref, rows, width, value):
    def body(r, _):
        for k in range(width // L):
            ref[r, pl.ds(k * L, L)] = jnp.full((L,), value, jnp.float32)
        return 0
    lax.fori_loop(0, rows, body, 0)


def _fill1d(ref, n, value):
    def body(i, _):
        ref[pl.ds(i * L, L)] = jnp.full((L,), value, jnp.float32)
        return 0
    lax.fori_loop(0, n // L, body, 0)


R2PT = R2 // 16


def _count_body(he_ref, out_ref, idx_v, ones_v, zer_v, cnt_n, cnt_e):
    cid = lax.axis_index("c")
    sid = lax.axis_index("s")
    _fill1d(ones_v, CHC, 1.0)
    _fill1d(zer_v, R2PT, 0.0)
    for g_local in range(2):
        g = cid * 2 + g_local
        pltpu.sync_copy(zer_v, cnt_n.at[pl.ds(sid * R2PT, R2PT)])
        pltpu.sync_copy(zer_v, cnt_e.at[pl.ds(sid * R2PT, R2PT)])
        plsc.subcore_barrier()
        for which, tbl in ((0, cnt_n), (1, cnt_e)):
            for c in range(NCHC):
                base = (g * 2 + which) * E + sid * EPT + c * CHC
                pltpu.sync_copy(he_ref.at[pl.ds(base, CHC)], idx_v)
                pltpu.sync_copy(ones_v, tbl.at[idx_v], add=True)
        plsc.subcore_barrier()
        pltpu.sync_copy(cnt_n.at[pl.ds(sid * R2PT, R2PT)], zer_v)
        pltpu.sync_copy(zer_v, out_ref.at[pl.ds((g * 2) * R2 + sid * R2PT, R2PT)])
        pltpu.sync_copy(cnt_e.at[pl.ds(sid * R2PT, R2PT)], zer_v)
        pltpu.sync_copy(zer_v, out_ref.at[pl.ds((g * 2 + 1) * R2 + sid * R2PT, R2PT)])
        _fill1d(zer_v, R2PT, 0.0)


def _sc_counts(he_flat):
    k = pl.kernel(
        _count_body,
        out_type=jax.ShapeDtypeStruct((G * 2 * R2,), jnp.float32),
        mesh=_MESH,
        scratch_types=[
            pltpu.VMEM((CHC,), jnp.int32),
            pltpu.VMEM((CHC,), jnp.float32),
            pltpu.VMEM((R2PT,), jnp.float32),
            pltpu.VMEM_SHARED((R2,), jnp.float32),
            pltpu.VMEM_SHARED((R2,), jnp.float32),
        ],
    )
    return k(he_flat)


_WB = tuple((j * 80, 80) for j in range(8))
NDB = 31
NEPI = EPT // CH - NDB * 8


def _scatter_body(w, relu, src_ref, idxs_ref, idxd_ref, cnt_ref, out_ref,
                  isb0, isb1, idb0, idb1, rows, inv_v, acc,
                  s_i0, s_i1, s_g, s_s):
    cid = lax.axis_index("c")
    sid = lax.axis_index("s")
    isb = (isb0, isb1)
    idb = (idb0, idb1)
    s_ip = (s_i0, s_i1)
    r0 = rows[0]

    def issue_idx(p, base):
        pltpu.async_copy(idxs_ref.at[pl.ds(base, 4 * CH)], isb[p], s_ip[p])
        for k in range(4):
            pltpu.async_copy(idxd_ref.at[pl.ds(base + k * CH, CH)],
                             idb[p][k], s_ip[p])

    def wait_idx(p, base):
        pltpu.make_async_copy(idxs_ref.at[pl.ds(base, 4 * CH)],
                              isb[p], s_ip[p]).wait()
        for k in range(4):
            pltpu.make_async_copy(idxd_ref.at[pl.ds(base + k * CH, CH)],
                                  idb[p][k], s_ip[p]).wait()

    def drain_step(p):
        for k in range(4):
            pltpu.make_async_copy(rows[k], acc.at[idb[p][k]], s_s[k]).wait()

    def chunks4(p):
        for k in range(4):
            ix = isb[p].at[pl.ds(k * CH, CH)]
            pltpu.async_copy(src_ref.at[ix], rows[k], s_g[k])
        for k in range(4):
            ix = isb[p].at[pl.ds(k * CH, CH)]
            pltpu.make_async_copy(src_ref.at[ix], rows[k], s_g[k]).wait()
            pltpu.async_copy(rows[k], acc.at[idb[p][k]], s_s[k], add=True)

    for g_local in range(2):
        g = cid * 2 + g_local
        gbase = g * E + sid * EPT
        issue_idx(0, gbase)
        _fill_loop(r0, CH, w, 0.0)
        for off, sz in _WB:
            pltpu.sync_copy(r0.at[pl.ds(0, sz)],
                            acc.at[pl.ds(sid * RPT + off, sz)])
        plsc.subcore_barrier()

        def dblock(db, _):
            cb = gbase + db * 8 * CH

            @pl.when(db > 0)
            def _():
                drain_step(1)
            wait_idx(0, cb)
            issue_idx(1, cb + 4 * CH)
            chunks4(0)
            drain_step(0)
            wait_idx(1, cb + 4 * CH)

            @pl.when(db < NDB - 1)
            def _():
                issue_idx(0, cb + 8 * CH)
            chunks4(1)
            return 0

        lax.fori_loop(0, NDB, dblock, 0)
        drain_step(1)

        def epi(c, _):
            eb = gbase + c * CH
            pltpu.sync_copy(idxs_ref.at[pl.ds(eb, CH)], idb0[0])
            pltpu.sync_copy(idxd_ref.at[pl.ds(eb, CH)], idb0[1])
            pltpu.sync_copy(src_ref.at[idb0[0]], r0)
            pltpu.sync_copy(r0, acc.at[idb0[1]], add=True)
            return 0

        lax.fori_loop(NDB * 8, NDB * 8 + NEPI, epi, 0)
        plsc.subcore_barrier()
        pltpu.sync_copy(cnt_ref.at[pl.ds(g * R + sid * RPT, RPT)], inv_v)

        def invloop(i, _):
            c = inv_v[pl.ds(i * L, L)]
            inv_v[pl.ds(i * L, L)] = jnp.where(
                c > 0, 1.0 / c, jnp.zeros((L,), jnp.float32))
            return 0

        lax.fori_loop(0, RPT // L, invloop, 0)
        for off, sz in _WB:
            wb = sid * RPT + off
            pltpu.sync_copy(acc.at[pl.ds(wb, sz)], r0.at[pl.ds(0, sz)])

            def rowscale(i16, _):
                invc = inv_v[pl.ds(off + i16 * L, L)]
                for rloc in range(L):
                    sc = invc[rloc]
                    rr = i16 * L + rloc
                    for j in range(w // L):
                        v = r0[rr, pl.ds(j * L, L)] * sc
                        if relu:
                            v = jnp.maximum(v, 0.0)
                        r0[rr, pl.ds(j * L, L)] = v
                return 0

            lax.fori_loop(0, sz // L, rowscale, 0)
            pltpu.sync_copy(r0.at[pl.ds(0, sz)],
                            out_ref.at[pl.ds(g * R + wb, sz)])


def _sc_scatter(src_flat, idxs, idxd, cnt_flat, relu, w=128):
    k = pl.kernel(
        functools.partial(_scatter_body, w, relu),
        out_type=jax.ShapeDtypeStruct((G * R, w), jnp.float32),
        mesh=_MESH,
        scratch_types=[
            pltpu.VMEM((4 * CH,), jnp.int32),
            pltpu.VMEM((4 * CH,), jnp.int32),
            [pltpu.VMEM((CH,), jnp.int32)] * 4,
            [pltpu.VMEM((CH,), jnp.int32)] * 4,
            [pltpu.VMEM((CH, w), jnp.float32)] * 4,
            pltpu.VMEM((RPT,), jnp.float32),
            pltpu.VMEM_SHARED((R, w), jnp.float32),
            pltpu.SemaphoreType.DMA,
            pltpu.SemaphoreType.DMA,
            [pltpu.SemaphoreType.DMA] * 4,
            [pltpu.SemaphoreType.DMA] * 4,
        ],
    )
    return k(src_flat, idxs, idxd, cnt_flat)


NGR = G * B
GPW = NGR // 32
CHG = 64


def _gather_body(x1_ref, x2_ref, e1_ref, e2_ref, mk_ref, ea_ref, ec_ref,
                 xg1_ref, xg2_ref, mn1_ref, mx1_ref, mn2_ref, mx2_ref,
                 mk_v, ea_v, ec_v, bx, b1a, b1c, b2a, b2c):
    cid = lax.axis_index("c")
    sid = lax.axis_index("s")
    wid = cid * 16 + sid
    for r in range(GPW // CHG):
        wb = wid * GPW + r * CHG
        pltpu.sync_copy(mk_ref.at[pl.ds(wb, CHG)], mk_v)
        pltpu.sync_copy(ea_ref.at[pl.ds(wb, CHG)], ea_v)
        pltpu.sync_copy(ec_ref.at[pl.ds(wb, CHG)], ec_v)
        pltpu.sync_copy(x1_ref.at[mk_v], bx)
        pltpu.sync_copy(bx, xg1_ref.at[pl.ds(wb, CHG)])
        pltpu.sync_copy(x2_ref.at[mk_v], bx)
        pltpu.sync_copy(bx, xg2_ref.at[pl.ds(wb, CHG)])
        pltpu.sync_copy(e1_ref.at[ea_v], b1a)
        pltpu.sync_copy(e1_ref.at[ec_v], b1c)
        pltpu.sync_copy(e2_ref.at[ea_v], b2a)
        pltpu.sync_copy(e2_ref.at[ec_v], b2c)

        def body(i, _):
            for k in range(128 // L):
                sl = pl.ds(k * L, L)
                a1 = b1a[i, sl]; c1 = b1c[i, sl]
                b1a[i, sl] = jnp.minimum(a1, c1)
                b1c[i, sl] = jnp.maximum(a1, c1)
                a2 = b2a[i, sl]; c2 = b2c[i, sl]
                b2a[i, sl] = jnp.minimum(a2, c2)
                b2c[i, sl] = jnp.maximum(a2, c2)
            return 0
        lax.fori_loop(0, CHG, body, 0)
        pltpu.sync_copy(b1a, mn1_ref.at[pl.ds(wb, CHG)])
        pltpu.sync_copy(b1c, mx1_ref.at[pl.ds(wb, CHG)])
        pltpu.sync_copy(b2a, mn2_ref.at[pl.ds(wb, CHG)])
        pltpu.sync_copy(b2c, mx2_ref.at[pl.ds(wb, CHG)])


def _sc_gather(x1a, x2a, e1a, e2a, mk, ea, ec):
    f32 = jnp.float32
    k = pl.kernel(
        _gather_body,
        out_type=[
            jax.ShapeDtypeStruct((NGR, 128), f32),
            jax.ShapeDtypeStruct((NGR, 128), f32),
            jax.ShapeDtypeStruct((NGR, 128), f32),
            jax.ShapeDtypeStruct((NGR, 128), f32),
            jax.ShapeDtypeStruct((NGR, 128), f32),
            jax.ShapeDtypeStruct((NGR, 128), f32),
        ],
        mesh=_MESH,
        scratch_types=[
            pltpu.VMEM((CHG,), jnp.int32),
            pltpu.VMEM((CHG,), jnp.int32),
            pltpu.VMEM((CHG,), jnp.int32),
            pltpu.VMEM((CHG, 128), f32),
            pltpu.VMEM((CHG, 128), f32),
            pltpu.VMEM((CHG, 128), f32),
            pltpu.VMEM((CHG, 128), f32),
            pltpu.VMEM((CHG, 128), f32),
        ],
    )
    return k(x1a, x2a, e1a, e2a, mk, ea, ec)


BR = 512


def _matmul_body(x_ref, w_ref, b_ref, o_ref):
    o_ref[...] = (jnp.dot(x_ref[0], w_ref[0],
                          preferred_element_type=jnp.float32)
                  + b_ref[0, 0][None])[None]


def _tc_matmul(xp, wst, bst):
    g4, r, din = xp.shape
    dout = wst.shape[-1]
    return pl.pallas_call(
        _matmul_body,
        grid=(g4, r // BR),
        in_specs=[
            pl.BlockSpec((1, BR, din), lambda g, i: (g, i, 0)),
            pl.BlockSpec((1, din, dout), lambda g, i: (g, 0, 0)),
            pl.BlockSpec((1, 1, dout), lambda g, i: (g, 0, 0)),
        ],
        out_specs=pl.BlockSpec((1, BR, dout), lambda g, i: (g, i, 0)),
        out_shape=jax.ShapeDtypeStruct((g4, r, dout), jnp.float32),
    )(xp, wst, bst.reshape(g4, 1, dout))


def _scale_body(relu, raw_ref, cnt_ref, o_ref):
    c = cnt_ref[0]
    inv = jnp.where(c > 0, 1.0 / c, 0.0)
    v = raw_ref[0] * inv
    if relu:
        v = jnp.maximum(v, 0.0)
    o_ref[...] = v[None]


def _tc_scale(raw, cnt, relu):
    g4, r, w = raw.shape
    return pl.pallas_call(
        functools.partial(_scale_body, relu),
        grid=(g4, r // BR),
        in_specs=[
            pl.BlockSpec((1, BR, w), lambda g, i: (g, i, 0)),
            pl.BlockSpec((1, BR, 1), lambda g, i: (g, i, 0)),
        ],
        out_specs=pl.BlockSpec((1, BR, w), lambda g, i: (g, i, 0)),
        out_shape=jax.ShapeDtypeStruct((g4, r, w), jnp.float32),
    )(raw, cnt.reshape(g4, r, 1))


def _head_body(mn1_ref, mn2_ref, mx1_ref, mx2_ref, xg1_ref, xg2_ref,
               pos_ref, wq_ref, wk_ref, wv_ref,
               w1_ref, b1_ref, w2_ref, b2_ref, out_ref):
    def feat(g):
        return jnp.concatenate(
            [mn1_ref[g][:, :64], mn2_ref[g][:, :64],
             mx1_ref[g][:, :64], mx2_ref[g][:, :64],
             xg1_ref[g], xg2_ref[g]], axis=1)

    xts = [feat(t) + pos_ref[t][None] for t in range(T)]
    col = feat(3)
    xt2 = xts[2]
    q2 = jnp.dot(xt2, wq_ref[...], preferred_element_type=jnp.float32)
    scale = 1.0 / jnp.sqrt(jnp.float32(FEAT))
    ss, vs = [], []
    for s in range(T):
        k_s = jnp.dot(xts[s], wk_ref[...], preferred_element_type=jnp.float32)
        v_s = jnp.dot(xts[s], wv_ref[...], preferred_element_type=jnp.float32)
        ss.append(jnp.sum(q2 * k_s, axis=1, keepdims=True) * scale)
        vs.append(v_s)
    m = jnp.maximum(jnp.maximum(ss[0], ss[1]), ss[2])
    es = [jnp.exp(s_ - m) for s_ in ss]
    denom = es[0] + es[1] + es[2]
    to = (es[0] * vs[0] + es[1] * vs[1] + es[2] * vs[2]) / denom + xt2
    out_cat = jnp.concatenate([to, col], axis=1)
    h = jnp.maximum(jnp.dot(out_cat, w1_ref[...],
                            preferred_element_type=jnp.float32)
                    + b1_ref[...], 0.0)
    logits = jnp.dot(h, w2_ref[...], preferred_element_type=jnp.float32) \
        + b2_ref[...]
    l0 = logits[:, 0:1]
    l1 = logits[:, 1:2]
    mx = jnp.maximum(l0, l1)
    lse = mx + jnp.log(jnp.exp(l0 - mx) + jnp.exp(l1 - mx))
    out_ref[...] = logits - lse


def _tc_head(mn1, mn2, mx1, mx2, xg1, xg2, pos, Wq, Wk, Wv, W1, b1, W2, b2):
    nb = 4
    bb = B // nb
    return pl.pallas_call(
        _head_body,
        grid=(nb,),
        in_specs=[
            pl.BlockSpec((G, bb, 128), lambda i: (0, i, 0)),
            pl.BlockSpec((G, bb, 128), lambda i: (0, i, 0)),
            pl.BlockSpec((G, bb, 128), lambda i: (0, i, 0)),
            pl.BlockSpec((G, bb, 128), lambda i: (0, i, 0)),
            pl.BlockSpec((G, bb, 128), lambda i: (0, i, 0)),
            pl.BlockSpec((G, bb, 128), lambda i: (0, i, 0)),
            pl.BlockSpec((T, FEAT), lambda i: (0, 0)),
            pl.BlockSpec((FEAT, FEAT), lambda i: (0, 0)),
            pl.BlockSpec((FEAT, FEAT), lambda i: (0, 0)),
            pl.BlockSpec((FEAT, FEAT), lambda i: (0, 0)),
            pl.BlockSpec((2 * FEAT, HID), lambda i: (0, 0)),
            pl.BlockSpec((HID,), lambda i: (0,)),
            pl.BlockSpec((HID, 2), lambda i: (0, 0)),
            pl.BlockSpec((2,), lambda i: (0,)),
        ],
        out_specs=pl.BlockSpec((bb, 2), lambda i: (i, 0)),
        out_shape=jax.ShapeDtypeStruct((B, 2), jnp.float32),
    )(mn1, mn2, mx1, mx2, xg1, xg2, pos, Wq, Wk, Wv, W1, b1, W2, b2)


def kernel(x, edge_x, hyperedge_index, marks, edge_marks, Wn0, bn0, Wn1, bn1,
           We0, be0, We1, be1, Wcn0, bcn0, Wcn1, bcn1, Wce0, bce0, Wce1, bce1,
           pos_emb, Wq, Wk, Wv, W1, b1, W2, b2):
    f32 = jnp.float32
    he = hyperedge_index.astype(jnp.int32)
    he_n = he[:, 0, :]
    he_e = he[:, 1, :]
    g_off = (jnp.arange(G, dtype=jnp.int32) * R)[:, None]
    idx_nR = (he_n + g_off).reshape(-1)
    idx_eR = (he_e + g_off).reshape(-1)
    dst_n = he_n.reshape(-1)
    dst_e = he_e.reshape(-1)
    he_flat = he.reshape(-1)

    mk_adj = (marks.astype(jnp.int32) + g_off).reshape(-1)
    ea_adj = (edge_marks.astype(jnp.int32) + g_off).reshape(-1)
    ec_adj = ea_adj + 1

    pad = jnp.zeros((G, R - N, x.shape[-1]), f32)
    xp = jnp.concatenate([x, pad], axis=1)
    pad_e = jnp.zeros((G, R - M, edge_x.shape[-1]), f32)
    exp_ = jnp.concatenate([edge_x, pad_e], axis=1)

    cnt = _sc_counts(he_flat).reshape(G, 2, R)
    cnt_n = cnt[:, 0].reshape(-1)
    cnt_e = cnt[:, 1].reshape(-1)

    def stack_w(w_t, w_c):
        return jnp.stack([w_t, w_t, w_t, w_c])

    Wn0s, bn0s = stack_w(Wn0, Wcn0), stack_w(bn0, bcn0)
    Wn1s, bn1s = stack_w(Wn1, Wcn1), stack_w(bn1, bcn1)

    def pad_w(w, rows, cols):
        return jnp.pad(w, ((0, rows - w.shape[0]), (0, cols - w.shape[1])))

    We0s = stack_w(pad_w(We0, 128, 128), pad_w(Wce0, 128, 128))
    be0s = stack_w(jnp.pad(be0, (0, 64)), jnp.pad(bce0, (0, 64)))
    We1s = stack_w(pad_w(We1, 128, 128), pad_w(Wce1, 128, 128))
    be1s = stack_w(jnp.pad(be1, (0, 64)), jnp.pad(bce1, (0, 64)))

    def layer(xact, eact, Wns, bns, Wes, bes):
        Xn = _tc_matmul(xact, Wns, bns)
        Xe = _tc_matmul(eact, Wes, bes)
        m = _sc_scatter(Xn.reshape(G * R, 128), idx_nR, dst_e, cnt_e, False)
        xn = _sc_scatter(m, idx_eR, dst_n, cnt_n, True)
        mc = _sc_scatter(Xe.reshape(G * R, 128), idx_eR, dst_n, cnt_n, False)
        en = _sc_scatter(mc, idx_nR, dst_e, cnt_e, True)
        return xn.reshape(G, R, 128), en.reshape(G, R, 128)

    x1a, e1a = layer(xp, exp_, Wn0s, bn0s, We0s, be0s)
    x2a, e2a = layer(x1a, e1a, Wn1s, bn1s, We1s, be1s)

    xg1, xg2, mn1, mx1, mn2, mx2 = _sc_gather(
        x1a.reshape(G * R, 128), x2a.reshape(G * R, 128),
        e1a.reshape(G * R, 128), e2a.reshape(G * R, 128),
        mk_adj, ea_adj, ec_adj)

    return _tc_head(mn1.reshape(G, B, 128), mn2.reshape(G, B, 128),
                    mx1.reshape(G, B, 128), mx2.reshape(G, B, 128),
                    xg1.reshape(G, B, 128), xg2.reshape(G, B, 128),
                    pos_emb, Wq, Wk, Wv, W1, b1, W2, b2)


# speedup vs baseline: 1.1020x; 1.1020x over previous
import functools

import jax
import jax.numpy as jnp
from jax import lax
from jax.experimental import pallas as pl
from jax.experimental.pallas import tpu as pltpu
from jax.experimental.pallas import tpu_sc as plsc

N = 10000
M = 10000
E = 320000
G = 4
T = 3
FEAT = 512
HID = 256
B = 2048

R = 10240
RPT = R // 16
R2 = R
EPT = E // 16
CH = 160
NPAIR = 62
CHC = 2000
NCHC = EPT // CHC
L = 16

_MESH = plsc.VectorSubcoreMesh(core_axis_name="c", subcore_axis_name="s",
                               num_cores=2, num_subcores=16)


def _fill_loop(ref, rows, width, value):
    def body(r, _):
        for k in range(width // L):
            ref[r, pl.ds(k * L, L)] = jnp.full((L,), value, jnp.float32)
        return 0
    lax.fori_loop(0, rows, body, 0)


def _fill1d(ref, n, value):
    def body(i, _):
        ref[pl.ds(i * L, L)] = jnp.full((L,), value, jnp.float32)
        return 0
    lax.fori_loop(0, n // L, body, 0)


R2PT = R2 // 16


def _count_body(he_ref, out_ref, idx_v, ones_v, zer_v, cnt_n, cnt_e):
    cid = lax.axis_index("c")
    sid = lax.axis_index("s")
    _fill1d(ones_v, CHC, 1.0)
    _fill1d(zer_v, R2PT, 0.0)
    for g_local in range(2):
        g = cid * 2 + g_local
        pltpu.sync_copy(zer_v, cnt_n.at[pl.ds(sid * R2PT, R2PT)])
        pltpu.sync_copy(zer_v, cnt_e.at[pl.ds(sid * R2PT, R2PT)])
        plsc.subcore_barrier()
        for which, tbl in ((0, cnt_n), (1, cnt_e)):
            for c in range(NCHC):
                base = (g * 2 + which) * E + sid * EPT + c * CHC
                pltpu.sync_copy(he_ref.at[pl.ds(base, CHC)], idx_v)
                pltpu.sync_copy(ones_v, tbl.at[idx_v], add=True)
        plsc.subcore_barrier()
        pltpu.sync_copy(cnt_n.at[pl.ds(sid * R2PT, R2PT)], zer_v)
        pltpu.sync_copy(zer_v, out_ref.at[pl.ds((g * 2) * R2 + sid * R2PT, R2PT)])
        pltpu.sync_copy(cnt_e.at[pl.ds(sid * R2PT, R2PT)], zer_v)
        pltpu.sync_copy(zer_v, out_ref.at[pl.ds((g * 2 + 1) * R2 + sid * R2PT, R2PT)])
        _fill1d(zer_v, R2PT, 0.0)


def _sc_counts(he_flat):
    k = pl.kernel(
        _count_body,
        out_type=jax.ShapeDtypeStruct((G * 2 * R2,), jnp.float32),
        mesh=_MESH,
        scratch_types=[
            pltpu.VMEM((CHC,), jnp.int32),
            pltpu.VMEM((CHC,), jnp.float32),
            pltpu.VMEM((R2PT,), jnp.float32),
            pltpu.VMEM_SHARED((R2,), jnp.float32),
            pltpu.VMEM_SHARED((R2,), jnp.float32),
        ],
    )
    return k(he_flat)


_WB = ((0, 160), (160, 160), (320, 160), (480, 160))
NDB = 15
NEPI = EPT // CH - NDB * 8


def _scatter_body(w, relu, src_ref, idxs_ref, idxd_ref, cnt_ref, out_ref,
                  isb0, isb1, idb0, idb1, r0, r1, inv_v, acc,
                  s_i0, s_i1, s_g0, s_g1, s_s0, s_s1):
    cid = lax.axis_index("c")
    sid = lax.axis_index("s")
    isb = (isb0, isb1)
    idb = (idb0, idb1)
    s_ip = (s_i0, s_i1)
    rows = (r0, r1)
    s_g = (s_g0, s_g1)
    s_s = (s_s0, s_s1)

    def issue_idx(p, base):
        pltpu.async_copy(idxs_ref.at[pl.ds(base, 4 * CH)], isb[p], s_ip[p])
        for k in range(4):
            pltpu.async_copy(idxd_ref.at[pl.ds(base + k * CH, CH)],
                             idb[p][k], s_ip[p])

    def wait_idx(p, base):
        pltpu.make_async_copy(idxs_ref.at[pl.ds(base, 4 * CH)],
                              isb[p], s_ip[p]).wait()
        for k in range(4):
            pltpu.make_async_copy(idxd_ref.at[pl.ds(base + k * CH, CH)],
                                  idb[p][k], s_ip[p]).wait()

    def drain_step(p):
        pltpu.make_async_copy(rows[0], acc.at[idb[p][2]], s_s[0]).wait()
        pltpu.make_async_copy(rows[1], acc.at[idb[p][3]], s_s[1]).wait()

    def chunks4(p):
        for k in range(4):
            s = k % 2
            ix = isb[p].at[pl.ds(k * CH, CH)]
            if k >= 2:
                pltpu.make_async_copy(rows[s], acc.at[idb[p][k - 2]],
                                      s_s[s]).wait()
            pltpu.async_copy(src_ref.at[ix], rows[s], s_g[s])
            pltpu.make_async_copy(src_ref.at[ix], rows[s], s_g[s]).wait()
            pltpu.async_copy(rows[s], acc.at[idb[p][k]], s_s[s], add=True)

    for g_local in range(2):
        g = cid * 2 + g_local
        gbase = g * E + sid * EPT
        issue_idx(0, gbase)
        _fill_loop(r0, CH, w, 0.0)
        for off, sz in _WB:
            pltpu.sync_copy(r0.at[pl.ds(0, sz)],
                            acc.at[pl.ds(sid * RPT + off, sz)])
        plsc.subcore_barrier()

        def dblock(db, _):
            cb = gbase + db * 8 * CH

            @pl.when(db > 0)
            def _():
                drain_step(1)
            wait_idx(0, cb)
            issue_idx(1, cb + 4 * CH)
            chunks4(0)
            drain_step(0)
            wait_idx(1, cb + 4 * CH)
            issue_idx(0, cb + 8 * CH)
            chunks4(1)
            return 0

        lax.fori_loop(0, NDB, dblock, 0)
        ebase = gbase + NDB * 8 * CH
        wait_idx(0, ebase)
        drain_step(1)
        chunks4(0)
        drain_step(0)
        eb = gbase + (NDB * 8 + 4) * CH
        pltpu.sync_copy(idxs_ref.at[pl.ds(eb, CH)], idb1[0])
        pltpu.sync_copy(idxd_ref.at[pl.ds(eb, CH)], idb1[1])
        pltpu.sync_copy(src_ref.at[idb1[0]], r0)
        pltpu.sync_copy(r0, acc.at[idb1[1]], add=True)
        plsc.subcore_barrier()
        pltpu.sync_copy(cnt_ref.at[pl.ds(g * R + sid * RPT, RPT)], inv_v)

        def invloop(i, _):
            c = inv_v[pl.ds(i * L, L)]
            inv_v[pl.ds(i * L, L)] = jnp.where(
                c > 0, 1.0 / c, jnp.zeros((L,), jnp.float32))
            return 0

        lax.fori_loop(0, RPT // L, invloop, 0)
        for off, sz in _WB:
            wb = sid * RPT + off
            pltpu.sync_copy(acc.at[pl.ds(wb, sz)], r0.at[pl.ds(0, sz)])

            def rowscale(i16, _):
                invc = inv_v[pl.ds(off + i16 * L, L)]
                for rloc in range(L):
                    sc = invc[rloc]
                    rr = i16 * L + rloc
                    for j in range(w // L):
                        v = r0[rr, pl.ds(j * L, L)] * sc
                        if relu:
                            v = jnp.maximum(v, 0.0)
                        r0[rr, pl.ds(j * L, L)] = v
                return 0

            lax.fori_loop(0, sz // L, rowscale, 0)
            pltpu.sync_copy(r0.at[pl.ds(0, sz)],
                            out_ref.at[pl.ds(g * R + wb, sz)])


def _sc_scatter(src_flat, idxs, idxd, cnt_flat, relu, w=128):
    k = pl.kernel(
        functools.partial(_scatter_body, w, relu),
        out_type=jax.ShapeDtypeStruct((G * R, w), jnp.float32),
        mesh=_MESH,
        scratch_types=[
            pltpu.VMEM((4 * CH,), jnp.int32),
            pltpu.VMEM((4 * CH,), jnp.int32),
            [pltpu.VMEM((CH,), jnp.int32)] * 4,
            [pltpu.VMEM((CH,), jnp.int32)] * 4,
            pltpu.VMEM((CH, w), jnp.float32),
            pltpu.VMEM((CH, w), jnp.float32),
            pltpu.VMEM((RPT,), jnp.float32),
            pltpu.VMEM_SHARED((R, w), jnp.float32),
            pltpu.SemaphoreType.DMA,
            pltpu.SemaphoreType.DMA,
            pltpu.SemaphoreType.DMA,
            pltpu.SemaphoreType.DMA,
            pltpu.SemaphoreType.DMA,
            pltpu.SemaphoreType.DMA,
        ],
    )
    return k(src_flat, idxs, idxd, cnt_flat)


NGR = G * B
GPW = NGR // 32
CHG = 64


def _gather_body(x1_ref, x2_ref, e1_ref, e2_ref, mk_ref, ea_ref, ec_ref,
                 xg1_ref, xg2_ref, mn1_ref, mx1_ref, mn2_ref, mx2_ref,
                 mk_v, ea_v, ec_v, bx, b1a, b1c, b2a, b2c):
    cid = lax.axis_index("c")
    sid = lax.axis_index("s")
    wid = cid * 16 + sid
    for r in range(GPW // CHG):
        wb = wid * GPW + r * CHG
        pltpu.sync_copy(mk_ref.at[pl.ds(wb, CHG)], mk_v)
        pltpu.sync_copy(ea_ref.at[pl.ds(wb, CHG)], ea_v)
        pltpu.sync_copy(ec_ref.at[pl.ds(wb, CHG)], ec_v)
        pltpu.sync_copy(x1_ref.at[mk_v], bx)
        pltpu.sync_copy(bx, xg1_ref.at[pl.ds(wb, CHG)])
        pltpu.sync_copy(x2_ref.at[mk_v], bx)
        pltpu.sync_copy(bx, xg2_ref.at[pl.ds(wb, CHG)])
        pltpu.sync_copy(e1_ref.at[ea_v], b1a)
        pltpu.sync_copy(e1_ref.at[ec_v], b1c)
        pltpu.sync_copy(e2_ref.at[ea_v], b2a)
        pltpu.sync_copy(e2_ref.at[ec_v], b2c)

        def body(i, _):
            for k in range(128 // L):
                sl = pl.ds(k * L, L)
                a1 = b1a[i, sl]; c1 = b1c[i, sl]
                b1a[i, sl] = jnp.minimum(a1, c1)
                b1c[i, sl] = jnp.maximum(a1, c1)
                a2 = b2a[i, sl]; c2 = b2c[i, sl]
                b2a[i, sl] = jnp.minimum(a2, c2)
                b2c[i, sl] = jnp.maximum(a2, c2)
            return 0
        lax.fori_loop(0, CHG, body, 0)
        pltpu.sync_copy(b1a, mn1_ref.at[pl.ds(wb, CHG)])
        pltpu.sync_copy(b1c, mx1_ref.at[pl.ds(wb, CHG)])
        pltpu.sync_copy(b2a, mn2_ref.at[pl.ds(wb, CHG)])
        pltpu.sync_copy(b2c, mx2_ref.at[pl.ds(wb, CHG)])


def _sc_gather(x1a, x2a, e1a, e2a, mk, ea, ec):
    f32 = jnp.float32
    k = pl.kernel(
        _gather_body,
        out_type=[
            jax.ShapeDtypeStruct((NGR, 128), f32),
            jax.ShapeDtypeStruct((NGR, 128), f32),
            jax.ShapeDtypeStruct((NGR, 128), f32),
            jax.ShapeDtypeStruct((NGR, 128), f32),
            jax.ShapeDtypeStruct((NGR, 128), f32),
            jax.ShapeDtypeStruct((NGR, 128), f32),
        ],
        mesh=_MESH,
        scratch_types=[
            pltpu.VMEM((CHG,), jnp.int32),
            pltpu.VMEM((CHG,), jnp.int32),
            pltpu.VMEM((CHG,), jnp.int32),
            pltpu.VMEM((CHG, 128), f32),
            pltpu.VMEM((CHG, 128), f32),
            pltpu.VMEM((CHG, 128), f32),
            pltpu.VMEM((CHG, 128), f32),
            pltpu.VMEM((CHG, 128), f32),
        ],
    )
    return k(x1a, x2a, e1a, e2a, mk, ea, ec)


BR = 512


def _matmul_body(x_ref, w_ref, b_ref, o_ref):
    o_ref[...] = (jnp.dot(x_ref[0], w_ref[0],
                          preferred_element_type=jnp.float32)
                  + b_ref[0, 0][None])[None]


def _tc_matmul(xp, wst, bst):
    g4, r, din = xp.shape
    dout = wst.shape[-1]
    return pl.pallas_call(
        _matmul_body,
        grid=(g4, r // BR),
        in_specs=[
            pl.BlockSpec((1, BR, din), lambda g, i: (g, i, 0)),
            pl.BlockSpec((1, din, dout), lambda g, i: (g, 0, 0)),
            pl.BlockSpec((1, 1, dout), lambda g, i: (g, 0, 0)),
        ],
        out_specs=pl.BlockSpec((1, BR, dout), lambda g, i: (g, i, 0)),
        out_shape=jax.ShapeDtypeStruct((g4, r, dout), jnp.float32),
    )(xp, wst, bst.reshape(g4, 1, dout))


def _scale_body(relu, raw_ref, cnt_ref, o_ref):
    c = cnt_ref[0]
    inv = jnp.where(c > 0, 1.0 / c, 0.0)
    v = raw_ref[0] * inv
    if relu:
        v = jnp.maximum(v, 0.0)
    o_ref[...] = v[None]


def _tc_scale(raw, cnt, relu):
    g4, r, w = raw.shape
    return pl.pallas_call(
        functools.partial(_scale_body, relu),
        grid=(g4, r // BR),
        in_specs=[
            pl.BlockSpec((1, BR, w), lambda g, i: (g, i, 0)),
            pl.BlockSpec((1, BR, 1), lambda g, i: (g, i, 0)),
        ],
        out_specs=pl.BlockSpec((1, BR, w), lambda g, i: (g, i, 0)),
        out_shape=jax.ShapeDtypeStruct((g4, r, w), jnp.float32),
    )(raw, cnt.reshape(g4, r, 1))


def _head_body(mn1_ref, mn2_ref, mx1_ref, mx2_ref, xg1_ref, xg2_ref,
               pos_ref, wq_ref, wk_ref, wv_ref,
               w1_ref, b1_ref, w2_ref, b2_ref, out_ref):
    def feat(g):
        return jnp.concatenate(
            [mn1_ref[g][:, :64], mn2_ref[g][:, :64],
             mx1_ref[g][:, :64], mx2_ref[g][:, :64],
             xg1_ref[g], xg2_ref[g]], axis=1)

    xts = [feat(t) + pos_ref[t][None] for t in range(T)]
    col = feat(3)
    xt2 = xts[2]
    q2 = jnp.dot(xt2, wq_ref[...], preferred_element_type=jnp.float32)
    scale = 1.0 / jnp.sqrt(jnp.float32(FEAT))
    ss, vs = [], []
    for s in range(T):
        k_s = jnp.dot(xts[s], wk_ref[...], preferred_element_type=jnp.float32)
        v_s = jnp.dot(xts[s], wv_ref[...], preferred_element_type=jnp.float32)
        ss.append(jnp.sum(q2 * k_s, axis=1, keepdims=True) * scale)
        vs.append(v_s)
    m = jnp.maximum(jnp.maximum(ss[0], ss[1]), ss[2])
    es = [jnp.exp(s_ - m) for s_ in ss]
    denom = es[0] + es[1] + es[2]
    to = (es[0] * vs[0] + es[1] * vs[1] + es[2] * vs[2]) / denom + xt2
    out_cat = jnp.concatenate([to, col], axis=1)
    h = jnp.maximum(jnp.dot(out_cat, w1_ref[...],
                            preferred_element_type=jnp.float32)
                    + b1_ref[...], 0.0)
    logits = jnp.dot(h, w2_ref[...], preferred_element_type=jnp.float32) \
        + b2_ref[...]
    l0 = logits[:, 0:1]
    l1 = logits[:, 1:2]
    mx = jnp.maximum(l0, l1)
    lse = mx + jnp.log(jnp.exp(l0 - mx) + jnp.exp(l1 - mx))
    out_ref[...] = logits - lse


def _tc_head(mn1, mn2, mx1, mx2, xg1, xg2, pos, Wq, Wk, Wv, W1, b1, W2, b2):
    nb = 4
    bb = B // nb
    return pl.pallas_call(
        _head_body,
        grid=(nb,),
        in_specs=[
            pl.BlockSpec((G, bb, 128), lambda i: (0, i, 0)),
            pl.BlockSpec((G, bb, 128), lambda i: (0, i, 0)),
            pl.BlockSpec((G, bb, 128), lambda i: (0, i, 0)),
            pl.BlockSpec((G, bb, 128), lambda i: (0, i, 0)),
            pl.BlockSpec((G, bb, 128), lambda i: (0, i, 0)),
            pl.BlockSpec((G, bb, 128), lambda i: (0, i, 0)),
            pl.BlockSpec((T, FEAT), lambda i: (0, 0)),
            pl.BlockSpec((FEAT, FEAT), lambda i: (0, 0)),
            pl.BlockSpec((FEAT, FEAT), lambda i: (0, 0)),
            pl.BlockSpec((FEAT, FEAT), lambda i: (0, 0)),
            pl.BlockSpec((2 * FEAT, HID), lambda i: (0, 0)),
            pl.BlockSpec((HID,), lambda i: (0,)),
            pl.BlockSpec((HID, 2), lambda i: (0, 0)),
            pl.BlockSpec((2,), lambda i: (0,)),
        ],
        out_specs=pl.BlockSpec((bb, 2), lambda i: (i, 0)),
        out_shape=jax.ShapeDtypeStruct((B, 2), jnp.float32),
    )(mn1, mn2, mx1, mx2, xg1, xg2, pos, Wq, Wk, Wv, W1, b1, W2, b2)


def kernel(x, edge_x, hyperedge_index, marks, edge_marks, Wn0, bn0, Wn1, bn1,
           We0, be0, We1, be1, Wcn0, bcn0, Wcn1, bcn1, Wce0, bce0, Wce1, bce1,
           pos_emb, Wq, Wk, Wv, W1, b1, W2, b2):
    f32 = jnp.float32
    he = hyperedge_index.astype(jnp.int32)
    he_n = he[:, 0, :]
    he_e = he[:, 1, :]
    g_off = (jnp.arange(G, dtype=jnp.int32) * R)[:, None]
    idx_nR = (he_n + g_off).reshape(-1)
    idx_eR = (he_e + g_off).reshape(-1)
    dst_n = he_n.reshape(-1)
    dst_e = he_e.reshape(-1)
    he_flat = he.reshape(-1)

    mk_adj = (marks.astype(jnp.int32) + g_off).reshape(-1)
    ea_adj = (edge_marks.astype(jnp.int32) + g_off).reshape(-1)
    ec_adj = ea_adj + 1

    pad = jnp.zeros((G, R - N, x.shape[-1]), f32)
    xp = jnp.concatenate([x, pad], axis=1)
    pad_e = jnp.zeros((G, R - M, edge_x.shape[-1]), f32)
    exp_ = jnp.concatenate([edge_x, pad_e], axis=1)

    cnt = _sc_counts(he_flat).reshape(G, 2, R)
    cnt_n = cnt[:, 0].reshape(-1)
    cnt_e = cnt[:, 1].reshape(-1)

    def stack_w(w_t, w_c):
        return jnp.stack([w_t, w_t, w_t, w_c])

    Wn0s, bn0s = stack_w(Wn0, Wcn0), stack_w(bn0, bcn0)
    Wn1s, bn1s = stack_w(Wn1, Wcn1), stack_w(bn1, bcn1)

    def pad_w(w, rows, cols):
        return jnp.pad(w, ((0, rows - w.shape[0]), (0, cols - w.shape[1])))

    We0s = stack_w(pad_w(We0, 128, 128), pad_w(Wce0, 128, 128))
    be0s = stack_w(jnp.pad(be0, (0, 64)), jnp.pad(bce0, (0, 64)))
    We1s = stack_w(pad_w(We1, 128, 128), pad_w(Wce1, 128, 128))
    be1s = stack_w(jnp.pad(be1, (0, 64)), jnp.pad(bce1, (0, 64)))

    def layer(xact, eact, Wns, bns, Wes, bes):
        Xn = _tc_matmul(xact, Wns, bns)
        Xe = _tc_matmul(eact, Wes, bes)
        m = _sc_scatter(Xn.reshape(G * R, 128), idx_nR, dst_e, cnt_e, False)
        xn = _sc_scatter(m, idx_eR, dst_n, cnt_n, True)
        mc = _sc_scatter(Xe.reshape(G * R, 128), idx_eR, dst_n, cnt_n, False)
        en = _sc_scatter(mc, idx_nR, dst_e, cnt_e, True)
        return xn.reshape(G, R, 128), en.reshape(G, R, 128)

    x1a, e1a = layer(xp, exp_, Wn0s, bn0s, We0s, be0s)
    x2a, e2a = layer(x1a, e1a, Wn1s, bn1s, We1s, be1s)

    xg1, xg2, mn1, mx1, mn2, mx2 = _sc_gather(
        x1a.reshape(G * R, 128), x2a.reshape(G * R, 128),
        e1a.reshape(G * R, 128), e2a.reshape(G * R, 128),
        mk_adj, ea_adj, ec_adj)

    return _tc_head(mn1.reshape(G, B, 128), mn2.reshape(G, B, 128),
                    mx1.reshape(G, B, 128), mx2.reshape(G, B, 128),
                    xg1.reshape(G, B, 128), xg2.reshape(G, B, 128),
                    pos_emb, Wq, Wk, Wv, W1, b1, W2, b2)
